# baseline (speedup 1.0000x reference)
import functools

import jax
import jax.numpy as jnp
from jax import lax
from jax.experimental import pallas as pl
from jax.experimental.pallas import tpu as pltpu
from jax.experimental.pallas import tpu_sc as plsc

_N = 16384
_P_DIM = 64
_Q_DIM = 64
_NUM_WORKERS = 32
_ROWS_PER_WORKER = _N // _NUM_WORKERS
_TC_BLOCK = 256


def _sc_gather(x0, v0, d, w):
    mesh = plsc.VectorSubcoreMesh(core_axis_name="c", subcore_axis_name="s")

    @functools.partial(
        pl.kernel,
        out_type=[
            jax.ShapeDtypeStruct((_N, _P_DIM), jnp.float32),
            jax.ShapeDtypeStruct((_N, _Q_DIM), jnp.float32),
        ],
        mesh=mesh,
        scratch_types=[
            pltpu.VMEM((_ROWS_PER_WORKER,), jnp.int32),
            pltpu.VMEM((_ROWS_PER_WORKER,), jnp.int32),
            pltpu.VMEM((_ROWS_PER_WORKER, _P_DIM), jnp.float32),
            pltpu.VMEM((_ROWS_PER_WORKER, _Q_DIM), jnp.float32),
            pltpu.SemaphoreType.DMA,
            pltpu.SemaphoreType.DMA,
        ],
        compiler_params=pltpu.CompilerParams(use_tc_tiling_on_sc=False),
    )
    def gather_kernel(x0_hbm, v0_hbm, d_hbm, w_hbm, outx_hbm, outw_hbm,
                      idx_d, idx_w, rows_x, rows_w, sem_x, sem_w):
        wid = lax.axis_index("s") * 2 + lax.axis_index("c")
        base = wid * _ROWS_PER_WORKER
        pltpu.sync_copy(d_hbm.at[pl.ds(base, _ROWS_PER_WORKER)], idx_d)
        pltpu.sync_copy(w_hbm.at[pl.ds(base, _ROWS_PER_WORKER)], idx_w)
        cx = pltpu.async_copy(x0_hbm.at[idx_d], rows_x, sem_x)
        cw = pltpu.async_copy(v0_hbm.at[idx_w], rows_w, sem_w)
        cx.wait()
        cw.wait()
        pltpu.sync_copy(rows_x, outx_hbm.at[pl.ds(base, _ROWS_PER_WORKER)])
        pltpu.sync_copy(rows_w, outw_hbm.at[pl.ds(base, _ROWS_PER_WORKER)])

    return gather_kernel(x0, v0, d, w)


def _tc_expand_body(x_ref, w_ref, o_ref):
    x = x_ref[...]
    w = w_ref[...]
    sx = jnp.sum(x * x, axis=1, keepdims=True)
    sw = jnp.sum(w * w, axis=1, keepdims=True)
    xs = x * lax.rsqrt(sx * sw)
    for j in range(_P_DIM):
        o_ref[:, _Q_DIM * j:_Q_DIM * (j + 1)] = xs[:, j:j + 1] * w


def _tc_expand(rows_x, rows_w):
    grid = _N // _TC_BLOCK
    return pl.pallas_call(
        _tc_expand_body,
        grid=(grid,),
        in_specs=[
            pl.BlockSpec((_TC_BLOCK, _P_DIM), lambda i: (i, 0)),
            pl.BlockSpec((_TC_BLOCK, _Q_DIM), lambda i: (i, 0)),
        ],
        out_specs=pl.BlockSpec((_TC_BLOCK, _P_DIM * _Q_DIM), lambda i: (i, 0)),
        out_shape=jax.ShapeDtypeStruct((_N, _P_DIM * _Q_DIM), jnp.float32),
        compiler_params=pltpu.CompilerParams(
            dimension_semantics=("arbitrary",),
        ),
    )(rows_x, rows_w)


@jax.jit
def kernel(d, w, x0, v0):
    rows_x, rows_w = _sc_gather(x0, v0, d, w)
    return _tc_expand(rows_x, rows_w)


# speedup vs baseline: 1.8899x; 1.8899x over previous
import functools

import jax
import jax.numpy as jnp
from jax import lax
from jax.experimental import pallas as pl
from jax.experimental.pallas import tpu as pltpu
from jax.experimental.pallas import tpu_sc as plsc

_N = 16384
_P_DIM = 64
_Q_DIM = 64
_NUM_WORKERS = 32
_ROWS_PER_WORKER = _N // _NUM_WORKERS
_TC_BLOCK = 256


def _sc_gather(x0, v0, d, w):
    mesh = plsc.VectorSubcoreMesh(core_axis_name="c", subcore_axis_name="s")

    @functools.partial(
        pl.kernel,
        out_type=[
            jax.ShapeDtypeStruct((_N, _P_DIM), jnp.float32),
            jax.ShapeDtypeStruct((_N, _Q_DIM), jnp.float32),
        ],
        mesh=mesh,
        scratch_types=[
            pltpu.VMEM((_ROWS_PER_WORKER,), jnp.int32),
            pltpu.VMEM((_ROWS_PER_WORKER,), jnp.int32),
            pltpu.VMEM((_ROWS_PER_WORKER, _P_DIM), jnp.float32),
            pltpu.VMEM((_ROWS_PER_WORKER, _Q_DIM), jnp.float32),
            pltpu.SemaphoreType.DMA,
            pltpu.SemaphoreType.DMA,
        ],
        compiler_params=pltpu.CompilerParams(use_tc_tiling_on_sc=False),
    )
    def gather_kernel(x0_hbm, v0_hbm, d_hbm, w_hbm, outx_hbm, outw_hbm,
                      idx_d, idx_w, rows_x, rows_w, sem_x, sem_w):
        wid = lax.axis_index("s") * 2 + lax.axis_index("c")
        base = wid * _ROWS_PER_WORKER
        pltpu.sync_copy(d_hbm.at[pl.ds(base, _ROWS_PER_WORKER)], idx_d)
        pltpu.sync_copy(w_hbm.at[pl.ds(base, _ROWS_PER_WORKER)], idx_w)
        cx = pltpu.async_copy(x0_hbm.at[idx_d], rows_x, sem_x)
        cw = pltpu.async_copy(v0_hbm.at[idx_w], rows_w, sem_w)
        cx.wait()
        cw.wait()
        pltpu.sync_copy(rows_x, outx_hbm.at[pl.ds(base, _ROWS_PER_WORKER)])
        pltpu.sync_copy(rows_w, outw_hbm.at[pl.ds(base, _ROWS_PER_WORKER)])

    return gather_kernel(x0, v0, d, w)


def _tc_expand_body(x_ref, w_ref, r_ref, o_ref):
    x = x_ref[...]
    w = w_ref[...]
    sx = jnp.sum(x * x, axis=1, keepdims=True)
    sw = jnp.sum(w * w, axis=1, keepdims=True)
    xs = x * lax.rsqrt(sx * sw)
    xrep = jnp.dot(xs, r_ref[...], preferred_element_type=jnp.float32)
    wtile = pltpu.repeat(w, _P_DIM, axis=1)
    o_ref[...] = xrep * wtile


def _tc_expand(rows_x, rows_w):
    grid = _N // _TC_BLOCK
    jm = jnp.arange(_P_DIM * _Q_DIM, dtype=jnp.int32) // _Q_DIM
    rmat = (jm[None, :] == jnp.arange(_P_DIM, dtype=jnp.int32)[:, None])
    rmat = rmat.astype(jnp.float32)
    return pl.pallas_call(
        _tc_expand_body,
        grid=(grid,),
        in_specs=[
            pl.BlockSpec((_TC_BLOCK, _P_DIM), lambda i: (i, 0)),
            pl.BlockSpec((_TC_BLOCK, _Q_DIM), lambda i: (i, 0)),
            pl.BlockSpec((_P_DIM, _P_DIM * _Q_DIM), lambda i: (0, 0)),
        ],
        out_specs=pl.BlockSpec((_TC_BLOCK, _P_DIM * _Q_DIM), lambda i: (i, 0)),
        out_shape=jax.ShapeDtypeStruct((_N, _P_DIM * _Q_DIM), jnp.float32),
        compiler_params=pltpu.CompilerParams(
            dimension_semantics=("arbitrary",),
        ),
    )(rows_x, rows_w, rmat)


@jax.jit
def kernel(d, w, x0, v0):
    rows_x, rows_w = _sc_gather(x0, v0, d, w)
    return _tc_expand(rows_x, rows_w)


# speedup vs baseline: 2.0440x; 1.0816x over previous
import functools

import jax
import jax.numpy as jnp
from jax import lax
from jax.experimental import pallas as pl
from jax.experimental.pallas import tpu as pltpu
from jax.experimental.pallas import tpu_sc as plsc

_N = 16384
_P_DIM = 64
_Q_DIM = 64
_NUM_WORKERS = 32
_ROWS_PER_WORKER = _N // _NUM_WORKERS
_TC_BLOCK = 512


def _sc_gather(x0, v0, d, w):
    mesh = plsc.VectorSubcoreMesh(core_axis_name="c", subcore_axis_name="s")

    @functools.partial(
        pl.kernel,
        out_type=[
            jax.ShapeDtypeStruct((_N, _P_DIM), jnp.float32),
            jax.ShapeDtypeStruct((_N, _Q_DIM), jnp.float32),
        ],
        mesh=mesh,
        scratch_types=[
            pltpu.VMEM((_ROWS_PER_WORKER,), jnp.int32),
            pltpu.VMEM((_ROWS_PER_WORKER,), jnp.int32),
            pltpu.VMEM((_ROWS_PER_WORKER, _P_DIM), jnp.float32),
            pltpu.VMEM((_ROWS_PER_WORKER, _Q_DIM), jnp.float32),
            pltpu.SemaphoreType.DMA,
            pltpu.SemaphoreType.DMA,
        ],
        compiler_params=pltpu.CompilerParams(use_tc_tiling_on_sc=False),
    )
    def gather_kernel(x0_hbm, v0_hbm, d_hbm, w_hbm, outx_hbm, outw_hbm,
                      idx_d, idx_w, rows_x, rows_w, sem_x, sem_w):
        wid = lax.axis_index("s") * 2 + lax.axis_index("c")
        base = wid * _ROWS_PER_WORKER
        pltpu.sync_copy(d_hbm.at[pl.ds(base, _ROWS_PER_WORKER)], idx_d)
        pltpu.sync_copy(w_hbm.at[pl.ds(base, _ROWS_PER_WORKER)], idx_w)
        cx = pltpu.async_copy(x0_hbm.at[idx_d], rows_x, sem_x)
        cw = pltpu.async_copy(v0_hbm.at[idx_w], rows_w, sem_w)
        cx.wait()
        cw.wait()
        pltpu.sync_copy(rows_x, outx_hbm.at[pl.ds(base, _ROWS_PER_WORKER)])
        pltpu.sync_copy(rows_w, outw_hbm.at[pl.ds(base, _ROWS_PER_WORKER)])

    return gather_kernel(x0, v0, d, w)


def _tc_expand_body(x_ref, w_ref, r_ref, o_ref):
    x = x_ref[...]
    w = w_ref[...]
    sx = jnp.sum(x * x, axis=1, keepdims=True)
    sw = jnp.sum(w * w, axis=1, keepdims=True)
    xs = x * lax.rsqrt(sx * sw)
    xrep = jnp.dot(xs, r_ref[...], preferred_element_type=jnp.float32)
    wtile = pltpu.repeat(w, _P_DIM, axis=1)
    o_ref[...] = xrep * wtile


def _tc_expand(rows_x, rows_w):
    grid = _N // _TC_BLOCK
    jm = jnp.arange(_P_DIM * _Q_DIM, dtype=jnp.int32) // _Q_DIM
    rmat = (jm[None, :] == jnp.arange(_P_DIM, dtype=jnp.int32)[:, None])
    rmat = rmat.astype(jnp.float32)
    return pl.pallas_call(
        _tc_expand_body,
        grid=(grid,),
        in_specs=[
            pl.BlockSpec((_TC_BLOCK, _P_DIM), lambda i: (i, 0)),
            pl.BlockSpec((_TC_BLOCK, _Q_DIM), lambda i: (i, 0)),
            pl.BlockSpec((_P_DIM, _P_DIM * _Q_DIM), lambda i: (0, 0)),
        ],
        out_specs=pl.BlockSpec((_TC_BLOCK, _P_DIM * _Q_DIM), lambda i: (i, 0)),
        out_shape=jax.ShapeDtypeStruct((_N, _P_DIM * _Q_DIM), jnp.float32),
        compiler_params=pltpu.CompilerParams(
            dimension_semantics=("arbitrary",),
        ),
    )(rows_x, rows_w, rmat)


@jax.jit
def kernel(d, w, x0, v0):
    rows_x, rows_w = _sc_gather(x0, v0, d, w)
    return _tc_expand(rows_x, rows_w)


# speedup vs baseline: 2.0466x; 1.0013x over previous
import functools

import jax
import jax.numpy as jnp
from jax import lax
from jax.experimental import pallas as pl
from jax.experimental.pallas import tpu as pltpu
from jax.experimental.pallas import tpu_sc as plsc

_N = 16384
_P_DIM = 64
_Q_DIM = 64
_NUM_WORKERS = 32
_ROWS_PER_WORKER = _N // _NUM_WORKERS
_TC_BLOCK = 1024


def _sc_gather(x0, v0, d, w):
    mesh = plsc.VectorSubcoreMesh(core_axis_name="c", subcore_axis_name="s")

    @functools.partial(
        pl.kernel,
        out_type=[
            jax.ShapeDtypeStruct((_N, _P_DIM), jnp.float32),
            jax.ShapeDtypeStruct((_N, _Q_DIM), jnp.float32),
        ],
        mesh=mesh,
        scratch_types=[
            pltpu.VMEM((_ROWS_PER_WORKER,), jnp.int32),
            pltpu.VMEM((_ROWS_PER_WORKER,), jnp.int32),
            pltpu.VMEM((_ROWS_PER_WORKER, _P_DIM), jnp.float32),
            pltpu.VMEM((_ROWS_PER_WORKER, _Q_DIM), jnp.float32),
            pltpu.SemaphoreType.DMA,
            pltpu.SemaphoreType.DMA,
        ],
        compiler_params=pltpu.CompilerParams(use_tc_tiling_on_sc=False),
    )
    def gather_kernel(x0_hbm, v0_hbm, d_hbm, w_hbm, outx_hbm, outw_hbm,
                      idx_d, idx_w, rows_x, rows_w, sem_x, sem_w):
        wid = lax.axis_index("s") * 2 + lax.axis_index("c")
        base = wid * _ROWS_PER_WORKER
        pltpu.sync_copy(d_hbm.at[pl.ds(base, _ROWS_PER_WORKER)], idx_d)
        pltpu.sync_copy(w_hbm.at[pl.ds(base, _ROWS_PER_WORKER)], idx_w)
        cx = pltpu.async_copy(x0_hbm.at[idx_d], rows_x, sem_x)
        cw = pltpu.async_copy(v0_hbm.at[idx_w], rows_w, sem_w)
        cx.wait()
        cw.wait()
        pltpu.sync_copy(rows_x, outx_hbm.at[pl.ds(base, _ROWS_PER_WORKER)])
        pltpu.sync_copy(rows_w, outw_hbm.at[pl.ds(base, _ROWS_PER_WORKER)])

    return gather_kernel(x0, v0, d, w)


def _tc_expand_body(x_ref, w_ref, r_ref, o_ref):
    x = x_ref[...]
    w = w_ref[...]
    sx = jnp.sum(x * x, axis=1, keepdims=True)
    sw = jnp.sum(w * w, axis=1, keepdims=True)
    xs = x * lax.rsqrt(sx * sw)
    xrep = jnp.dot(xs, r_ref[...], preferred_element_type=jnp.float32)
    wtile = pltpu.repeat(w, _P_DIM, axis=1)
    o_ref[...] = xrep * wtile


def _tc_expand(rows_x, rows_w):
    grid = _N // _TC_BLOCK
    jm = jnp.arange(_P_DIM * _Q_DIM, dtype=jnp.int32) // _Q_DIM
    rmat = (jm[None, :] == jnp.arange(_P_DIM, dtype=jnp.int32)[:, None])
    rmat = rmat.astype(jnp.float32)
    return pl.pallas_call(
        _tc_expand_body,
        grid=(grid,),
        in_specs=[
            pl.BlockSpec((_TC_BLOCK, _P_DIM), lambda i: (i, 0)),
            pl.BlockSpec((_TC_BLOCK, _Q_DIM), lambda i: (i, 0)),
            pl.BlockSpec((_P_DIM, _P_DIM * _Q_DIM), lambda i: (0, 0)),
        ],
        out_specs=pl.BlockSpec((_TC_BLOCK, _P_DIM * _Q_DIM), lambda i: (i, 0)),
        out_shape=jax.ShapeDtypeStruct((_N, _P_DIM * _Q_DIM), jnp.float32),
        compiler_params=pltpu.CompilerParams(
            dimension_semantics=("arbitrary",),
        ),
    )(rows_x, rows_w, rmat)


@jax.jit
def kernel(d, w, x0, v0):
    rows_x, rows_w = _sc_gather(x0, v0, d, w)
    return _tc_expand(rows_x, rows_w)
